# initial kernel scaffold (unmeasured)
import jax
import jax.numpy as jnp
from jax import lax
from jax.experimental import pallas as pl
from jax.experimental.pallas import tpu as pltpu


def kernel(
    x,
):
    def body(*refs):
        pass

    out_shape = jax.ShapeDtypeStruct(..., jnp.float32)
    return pl.pallas_call(body, out_shape=out_shape)(...)


# baseline (device time: 10353 ns/iter reference)
import jax
import jax.numpy as jnp
from jax import lax
from jax.experimental import pallas as pl
from jax.experimental.pallas import tpu as pltpu

N_DEV = 4


def kernel(x):
    m_per, n = x.shape

    def body(x_ref, out_ref, comm_ref, send_sems, recv_sems):
        my_pos = lax.axis_index("i")
        left = lax.rem(my_pos + N_DEV - 1, N_DEV)
        right = lax.rem(my_pos + 1, N_DEV)

        barrier_sem = pltpu.get_barrier_semaphore()
        for nbr in (left, right):
            pl.semaphore_signal(
                barrier_sem, inc=1,
                device_id=(nbr,), device_id_type=pl.DeviceIdType.MESH,
            )
        pl.semaphore_wait(barrier_sem, 2)

        xv = x_ref[:, :]
        val = jnp.max(xv, axis=0, keepdims=True)
        rows = lax.broadcasted_iota(jnp.int32, (m_per, n), 0)
        idx_i = jnp.min(
            jnp.where(xv == val, rows, jnp.int32(2**30)),
            axis=0, keepdims=True,
        ) + my_pos * m_per
        idx = idx_i.astype(jnp.float32)
        comm_ref[0, 0:1, :] = val
        comm_ref[0, 1:2, :] = idx
        acc_val, acc_idx = val, idx

        for h in range(N_DEV - 1):
            rdma = pltpu.make_async_remote_copy(
                src_ref=comm_ref.at[h],
                dst_ref=comm_ref.at[h + 1],
                send_sem=send_sems.at[h],
                recv_sem=recv_sems.at[h],
                device_id=(right,),
                device_id_type=pl.DeviceIdType.MESH,
            )
            rdma.start()
            rdma.wait()

            rv = comm_ref[h + 1, 0:1, :]
            ri = comm_ref[h + 1, 1:2, :]
            take = (rv > acc_val) | ((rv == acc_val) & (ri < acc_idx))
            acc_val = jnp.where(take, rv, acc_val)
            acc_idx = jnp.where(take, ri, acc_idx)

        out_ref[0:1, :] = acc_val
        out_ref[1:2, :] = acc_idx

    return pl.pallas_call(
        body,
        out_shape=jax.ShapeDtypeStruct((2, n), jnp.float32),
        in_specs=[pl.BlockSpec(memory_space=pltpu.VMEM)],
        out_specs=pl.BlockSpec(memory_space=pltpu.VMEM),
        scratch_shapes=[
            pltpu.VMEM((N_DEV, 2, n), jnp.float32),
            pltpu.SemaphoreType.DMA((N_DEV - 1,)),
            pltpu.SemaphoreType.DMA((N_DEV - 1,)),
        ],
        compiler_params=pltpu.CompilerParams(collective_id=0),
    )(x)


# device time: 6785 ns/iter; 1.5259x vs baseline; 1.5259x over previous
import jax
import jax.numpy as jnp
from jax import lax
from jax.experimental import pallas as pl
from jax.experimental.pallas import tpu as pltpu

N_DEV = 4


def kernel(x):
    m_per, n = x.shape

    def body(x_ref, out_ref, comm_ref, send_sems, recv_sems):
        my_pos = lax.axis_index("i")

        barrier_sem = pltpu.get_barrier_semaphore()
        for d in range(1, N_DEV):
            pl.semaphore_signal(
                barrier_sem, inc=1,
                device_id=(lax.rem(my_pos + d, N_DEV),),
                device_id_type=pl.DeviceIdType.MESH,
            )

        xv = x_ref[:, :]
        val = jnp.max(xv, axis=0, keepdims=True)
        rows = lax.broadcasted_iota(jnp.int32, (m_per, n), 0)
        idx_i = jnp.min(
            jnp.where(xv == val, rows, jnp.int32(2**30)),
            axis=0, keepdims=True,
        ) + my_pos * m_per
        idx = idx_i.astype(jnp.float32)
        comm_ref[0, 0:1, :] = val
        comm_ref[0, 1:2, :] = idx

        pl.semaphore_wait(barrier_sem, N_DEV - 1)

        rdmas = []
        for d in range(1, N_DEV):
            rdma = pltpu.make_async_remote_copy(
                src_ref=comm_ref.at[0],
                dst_ref=comm_ref.at[d],
                send_sem=send_sems.at[d - 1],
                recv_sem=recv_sems.at[d - 1],
                device_id=(lax.rem(my_pos + d, N_DEV),),
                device_id_type=pl.DeviceIdType.MESH,
            )
            rdma.start()
            rdmas.append(rdma)

        acc_val, acc_idx = val, idx
        for d in (1, 3, 2):
            rdmas[d - 1].wait_recv()
            rv = comm_ref[d, 0:1, :]
            ri = comm_ref[d, 1:2, :]
            take = (rv > acc_val) | ((rv == acc_val) & (ri < acc_idx))
            acc_val = jnp.where(take, rv, acc_val)
            acc_idx = jnp.where(take, ri, acc_idx)

        out_ref[0:1, :] = acc_val
        out_ref[1:2, :] = acc_idx

        for rdma in rdmas:
            rdma.wait_send()

    return pl.pallas_call(
        body,
        out_shape=jax.ShapeDtypeStruct((2, n), jnp.float32),
        in_specs=[pl.BlockSpec(memory_space=pltpu.VMEM)],
        out_specs=pl.BlockSpec(memory_space=pltpu.VMEM),
        scratch_shapes=[
            pltpu.VMEM((N_DEV, 2, n), jnp.float32),
            pltpu.SemaphoreType.DMA((N_DEV - 1,)),
            pltpu.SemaphoreType.DMA((N_DEV - 1,)),
        ],
        compiler_params=pltpu.CompilerParams(collective_id=0),
    )(x)


# device time: 6771 ns/iter; 1.5290x vs baseline; 1.0021x over previous
import jax
import jax.numpy as jnp
from jax import lax
from jax.experimental import pallas as pl
from jax.experimental.pallas import tpu as pltpu

N_DEV = 4


def kernel(x):
    m_per, n = x.shape

    def body(x_ref, out_ref, comm_ref, send_sems, recv_sems):
        my_pos = lax.axis_index("i")

        barrier_sem = pltpu.get_barrier_semaphore()
        for d in range(1, N_DEV):
            pl.semaphore_signal(
                barrier_sem, inc=1,
                device_id=(lax.rem(my_pos + d, N_DEV),),
                device_id_type=pl.DeviceIdType.MESH,
            )

        xv = x_ref[:, :]
        val = jnp.max(xv, axis=0, keepdims=True)
        rows = lax.broadcasted_iota(jnp.int32, (m_per, n), 0)
        idx_i = jnp.min(
            jnp.where(xv == val, rows, jnp.int32(2**30)),
            axis=0, keepdims=True,
        ) + my_pos * m_per
        idx = idx_i.astype(jnp.float32)
        comm_ref[0, 0:1, :] = val
        comm_ref[0, 1:2, :] = idx

        pl.semaphore_wait(barrier_sem, N_DEV - 1)

        rdmas = []
        for d in range(1, N_DEV):
            rdma = pltpu.make_async_remote_copy(
                src_ref=comm_ref.at[0],
                dst_ref=comm_ref.at[d],
                send_sem=send_sems.at[d - 1],
                recv_sem=recv_sems.at[d - 1],
                device_id=(lax.rem(my_pos + d, N_DEV),),
                device_id_type=pl.DeviceIdType.MESH,
            )
            rdma.start()
            rdmas.append(rdma)

        acc_val, acc_idx = val, idx
        for d in (1, 3, 2):
            rdmas[d - 1].wait()
            rv = comm_ref[d, 0:1, :]
            ri = comm_ref[d, 1:2, :]
            take = (rv > acc_val) | ((rv == acc_val) & (ri < acc_idx))
            acc_val = jnp.where(take, rv, acc_val)
            acc_idx = jnp.where(take, ri, acc_idx)

        out_ref[0:1, :] = acc_val
        out_ref[1:2, :] = acc_idx

    return pl.pallas_call(
        body,
        out_shape=jax.ShapeDtypeStruct((2, n), jnp.float32),
        in_specs=[pl.BlockSpec(memory_space=pltpu.VMEM)],
        out_specs=pl.BlockSpec(memory_space=pltpu.VMEM),
        scratch_shapes=[
            pltpu.VMEM((N_DEV, 2, n), jnp.float32),
            pltpu.SemaphoreType.DMA((N_DEV - 1,)),
            pltpu.SemaphoreType.DMA((N_DEV - 1,)),
        ],
        compiler_params=pltpu.CompilerParams(collective_id=0),
    )(x)
